# baseline (speedup 1.0000x reference)
import functools

import jax
import jax.numpy as jnp
from jax import lax
from jax.experimental import pallas as pl
from jax.experimental.pallas import tpu as pltpu
from jax.experimental.pallas import tpu_sc as plsc

_info = plsc.get_sparse_core_info()
_NC, _NS = _info.num_cores, _info.num_subcores
_NW = _NC * _NS

_L = 16
_SUB = 4


def _fire(Wt_hbm, v, lane0, buf, sem):
    for j in range(_SUB):
        n = v[lane0 + j]
        o = pl.multiple_of((n >> 7) << 7, 128)
        pltpu.async_copy(
            Wt_hbm.at[:, pl.ds(o, 128)], buf.at[:, pl.ds(j * 128, 128)], sem
        )


def _drain(Wt_hbm, buf, sem):
    for j in range(_SUB):
        pltpu.make_async_copy(
            Wt_hbm.at[:, pl.ds(0, 128)], buf.at[:, pl.ds(j * 128, 128)], sem
        ).wait()


def _process(v, lane0, out_col0, buf, T):
    rows = lax.iota(jnp.int32, _L)
    for j in range(_SUB):
        r = v[lane0 + j] & 127
        col = jnp.full((_L,), j * 128, jnp.int32) + r
        out_col = jnp.full((_L,), out_col0 + j, jnp.int32)
        for t in range(4):
            vals = plsc.load_gather(buf, [rows + t * _L, col])
            plsc.store_scatter(T, [rows + t * _L, out_col], vals)


@functools.partial(jax.jit, static_argnames=("batch", "dim"))
def _gather(node_id, Wt, *, batch, dim):
    b_per_w = batch // _NW
    n_blocks = b_per_w // _L
    mesh = plsc.VectorSubcoreMesh(core_axis_name="c", subcore_axis_name="s")

    @functools.partial(
        pl.kernel,
        out_type=jax.ShapeDtypeStruct((dim, batch), jnp.float32),
        mesh=mesh,
        scratch_types=[
            pltpu.VMEM((b_per_w + _L,), jnp.int32),
            pltpu.VMEM((dim, _SUB * 128), jnp.float32),
            pltpu.VMEM((dim, _SUB * 128), jnp.float32),
            pltpu.VMEM((dim, b_per_w), jnp.float32),
            pltpu.SemaphoreType.DMA,
            pltpu.SemaphoreType.DMA,
        ],
        compiler_params=pltpu.CompilerParams(
            disable_bounds_checks=True, needs_layout_passes=False
        ),
    )
    def k(idx_hbm, Wt_hbm, out_hbm, idx_v, buf0, buf1, T, semA, semB):
        wid = lax.axis_index("s") * _NC + lax.axis_index("c")
        base = wid * b_per_w
        pltpu.sync_copy(idx_hbm.at[pl.ds(base, b_per_w)], idx_v.at[pl.ds(0, b_per_w)])
        idx_v[pl.ds(b_per_w, _L)] = jnp.zeros((_L,), jnp.int32)

        v0 = idx_v[pl.ds(0, _L)]
        _fire(Wt_hbm, v0, 0, buf0, semA)

        def body(blk, v):
            c0 = blk * _L
            _fire(Wt_hbm, v, _SUB, buf1, semB)
            _drain(Wt_hbm, buf0, semA)
            _process(v, 0, c0, buf0, T)

            _fire(Wt_hbm, v, 2 * _SUB, buf0, semA)
            _drain(Wt_hbm, buf1, semB)
            _process(v, _SUB, c0 + _SUB, buf1, T)

            _fire(Wt_hbm, v, 3 * _SUB, buf1, semB)
            _drain(Wt_hbm, buf0, semA)
            _process(v, 2 * _SUB, c0 + 2 * _SUB, buf0, T)

            v_next = idx_v[pl.ds((blk + 1) * _L, _L)]

            @pl.when(blk + 1 < n_blocks)
            def _():
                _fire(Wt_hbm, v_next, 0, buf0, semA)

            _drain(Wt_hbm, buf1, semB)
            _process(v, 3 * _SUB, c0 + 3 * _SUB, buf1, T)
            return v_next

        lax.fori_loop(0, n_blocks, body, v0, unroll=False)
        pltpu.sync_copy(T, out_hbm.at[:, pl.ds(base, b_per_w)])

    return k(node_id, Wt)


_GRP = 64


def _tc_body(idx_sref, idx3_ref, Wt_ref, out_ref, bufs, sems):
    grid = pl.num_programs(0)
    g = pl.program_id(0)
    slot = lax.rem(g, 2)
    nslot = lax.rem(g + 1, 2)

    def fire(step, s):
        for j in range(_GRP):
            n = idx_sref[step * _GRP + j]
            o = pl.multiple_of((n >> 7) << 7, 128)
            pltpu.make_async_copy(
                Wt_ref.at[:, pl.ds(o, 128)], bufs.at[s, j], sems.at[s, j % 4]
            ).start()

    @pl.when(g == 0)
    def _():
        fire(0, 0)

    @pl.when(g + 1 < grid)
    def _():
        fire(g + 1, nslot)

    for j in range(_GRP):
        pltpu.make_async_copy(
            Wt_ref.at[:, pl.ds(0, 128)], bufs.at[slot, j], sems.at[slot, j % 4]
        ).wait()

    dim = out_ref.shape[2]
    r = idx3_ref[0, 0, :] & 127
    sub = 16
    eye = (
        lax.broadcasted_iota(jnp.int32, (sub, 1, sub), 0)
        == lax.broadcasted_iota(jnp.int32, (sub, 1, sub), 2)
    ).astype(jnp.float32)
    for s in range(_GRP // sub):
        r_s = r[s * sub : (s + 1) * sub]
        Et = (
            lax.broadcasted_iota(jnp.int32, (128, sub), 0) == r_s[None, :]
        ).astype(jnp.float32)
        big = bufs[slot, pl.ds(s * sub, sub)].reshape(sub * dim, 128)
        C = lax.dot_general(
            big, Et, (((1,), (0,)), ((), ())), preferred_element_type=jnp.float32
        )
        D = C.reshape(sub, dim, sub)
        out_ref[0, pl.ds(s * sub, sub)] = jnp.sum(D * eye, axis=2)


@functools.partial(jax.jit, static_argnames=("batch", "dim"))
def _tc_gather(idx, Wt, *, batch, dim):
    grid = batch // _GRP
    idx3 = idx.reshape(grid, 1, _GRP)
    grid_spec = pltpu.PrefetchScalarGridSpec(
        num_scalar_prefetch=1,
        grid=(grid,),
        in_specs=[
            pl.BlockSpec((1, 1, _GRP), lambda g, sref: (g, 0, 0)),
            pl.BlockSpec(memory_space=pltpu.HBM),
        ],
        out_specs=pl.BlockSpec((1, _GRP, dim), lambda g, sref: (g, 0, 0)),
        scratch_shapes=[
            pltpu.VMEM((2, _GRP, dim, 128), jnp.float32),
            pltpu.SemaphoreType.DMA((2, 4)),
        ],
    )
    out = pl.pallas_call(
        _tc_body,
        grid_spec=grid_spec,
        out_shape=jax.ShapeDtypeStruct((grid, _GRP, dim), jnp.float32),
        compiler_params=pltpu.CompilerParams(disable_bounds_checks=True),
    )(idx, idx3, Wt)
    return out.reshape(batch, dim)


_SC_FRAC_NUM, _SC_FRAC_DEN = 1, 2


def kernel(g, node_id_copy, W):
    batch = node_id_copy.shape[0]
    dim = W.shape[1]
    b_sc = (batch * _SC_FRAC_NUM // _SC_FRAC_DEN) // _NW * _NW
    Wt = W.T
    out_sc_t = _gather(node_id_copy[:b_sc], Wt, batch=b_sc, dim=dim)
    out_tc = _tc_gather(node_id_copy[b_sc:], Wt, batch=batch - b_sc, dim=dim)
    return jnp.concatenate([out_sc_t.T, out_tc], axis=0)


# speedup vs baseline: 1.0172x; 1.0172x over previous
import functools

import jax
import jax.numpy as jnp
from jax import lax
from jax.experimental import pallas as pl
from jax.experimental.pallas import tpu as pltpu
from jax.experimental.pallas import tpu_sc as plsc

_info = plsc.get_sparse_core_info()
_NC, _NS = _info.num_cores, _info.num_subcores
_NW = _NC * _NS

_L = 16
_SUB = 4


def _fire(Wt_hbm, v, lane0, buf, sem):
    for j in range(_SUB):
        n = v[lane0 + j]
        o = pl.multiple_of((n >> 7) << 7, 128)
        pltpu.async_copy(
            Wt_hbm.at[:, pl.ds(o, 128)], buf.at[:, pl.ds(j * 128, 128)], sem
        )


def _drain(Wt_hbm, buf, sem):
    for j in range(_SUB):
        pltpu.make_async_copy(
            Wt_hbm.at[:, pl.ds(0, 128)], buf.at[:, pl.ds(j * 128, 128)], sem
        ).wait()


def _process(v, lane0, out_col0, buf, T):
    rows = lax.iota(jnp.int32, _L)
    for j in range(_SUB):
        r = v[lane0 + j] & 127
        col = jnp.full((_L,), j * 128, jnp.int32) + r
        out_col = jnp.full((_L,), out_col0 + j, jnp.int32)
        for t in range(4):
            vals = plsc.load_gather(buf, [rows + t * _L, col])
            plsc.store_scatter(T, [rows + t * _L, out_col], vals)


@functools.partial(jax.jit, static_argnames=("b_sc", "dim"))
def _gather(node_id, Wt, *, b_sc, dim):
    chunks_total = b_sc // 128
    base_chunks = chunks_total // _NW
    k_extra = chunks_total - base_chunks * _NW
    max_chunks = base_chunks + (1 if k_extra else 0)
    max_b = max_chunks * 128
    mesh = plsc.VectorSubcoreMesh(core_axis_name="c", subcore_axis_name="s")

    @functools.partial(
        pl.kernel,
        out_type=jax.ShapeDtypeStruct((dim, b_sc), jnp.float32),
        mesh=mesh,
        scratch_types=[
            pltpu.VMEM((max_b + _L,), jnp.int32),
            pltpu.VMEM((dim, _SUB * 128), jnp.float32),
            pltpu.VMEM((dim, _SUB * 128), jnp.float32),
            pltpu.VMEM((dim, max_b), jnp.float32),
            pltpu.SemaphoreType.DMA,
            pltpu.SemaphoreType.DMA,
        ],
        compiler_params=pltpu.CompilerParams(
            disable_bounds_checks=True, needs_layout_passes=False
        ),
    )
    def k(idx_hbm, Wt_hbm, out_hbm, idx_v, buf0, buf1, T, semA, semB):
        wid = lax.axis_index("s") * _NC + lax.axis_index("c")
        my_chunks = jnp.where(wid < k_extra, base_chunks + 1, base_chunks)
        n_blocks = my_chunks * (128 // _L)
        base = 128 * (base_chunks * wid + jnp.minimum(wid, k_extra))
        pltpu.sync_copy(
            idx_hbm.at[pl.ds(base, max_b)], idx_v.at[pl.ds(0, max_b)]
        )
        idx_v[pl.ds(max_b, _L)] = jnp.zeros((_L,), jnp.int32)

        v0 = idx_v[pl.ds(0, _L)]
        _fire(Wt_hbm, v0, 0, buf0, semA)

        def body(blk, v):
            c0 = blk * _L
            _fire(Wt_hbm, v, _SUB, buf1, semB)
            _drain(Wt_hbm, buf0, semA)
            _process(v, 0, c0, buf0, T)

            _fire(Wt_hbm, v, 2 * _SUB, buf0, semA)
            _drain(Wt_hbm, buf1, semB)
            _process(v, _SUB, c0 + _SUB, buf1, T)

            _fire(Wt_hbm, v, 3 * _SUB, buf1, semB)
            _drain(Wt_hbm, buf0, semA)
            _process(v, 2 * _SUB, c0 + 2 * _SUB, buf0, T)

            v_next = idx_v[pl.ds((blk + 1) * _L, _L)]

            @pl.when(blk + 1 < n_blocks)
            def _():
                _fire(Wt_hbm, v_next, 0, buf0, semA)

            _drain(Wt_hbm, buf1, semB)
            _process(v, 3 * _SUB, c0 + 3 * _SUB, buf1, T)
            return v_next

        lax.fori_loop(0, n_blocks, body, v0, unroll=False)
        for c in range(max_chunks):
            @pl.when(c < my_chunks)
            def _():
                col = pl.multiple_of(base + c * 128, 128)
                pltpu.sync_copy(
                    T.at[:, pl.ds(c * 128, 128)],
                    out_hbm.at[:, pl.ds(col, 128)],
                )

    return k(node_id, Wt)


_GRP = 64


def _tc_body(idx_sref, idx3_ref, Wt_ref, out_ref, bufs, sems):
    grid = pl.num_programs(0)
    g = pl.program_id(0)
    slot = lax.rem(g, 2)
    nslot = lax.rem(g + 1, 2)

    def fire(step, s):
        for j in range(_GRP):
            n = idx_sref[step * _GRP + j]
            o = pl.multiple_of((n >> 7) << 7, 128)
            pltpu.make_async_copy(
                Wt_ref.at[:, pl.ds(o, 128)], bufs.at[s, j], sems.at[s, j % 4]
            ).start()

    @pl.when(g == 0)
    def _():
        fire(0, 0)

    @pl.when(g + 1 < grid)
    def _():
        fire(g + 1, nslot)

    for j in range(_GRP):
        pltpu.make_async_copy(
            Wt_ref.at[:, pl.ds(0, 128)], bufs.at[slot, j], sems.at[slot, j % 4]
        ).wait()

    dim = out_ref.shape[2]
    r = idx3_ref[0, 0, :] & 127
    sub = 16
    eye = (
        lax.broadcasted_iota(jnp.int32, (sub, 1, sub), 0)
        == lax.broadcasted_iota(jnp.int32, (sub, 1, sub), 2)
    ).astype(jnp.float32)
    for s in range(_GRP // sub):
        r_s = r[s * sub : (s + 1) * sub]
        Et = (
            lax.broadcasted_iota(jnp.int32, (128, sub), 0) == r_s[None, :]
        ).astype(jnp.float32)
        big = bufs[slot, pl.ds(s * sub, sub)].reshape(sub * dim, 128)
        C = lax.dot_general(
            big, Et, (((1,), (0,)), ((), ())), preferred_element_type=jnp.float32
        )
        D = C.reshape(sub, dim, sub)
        out_ref[0, pl.ds(s * sub, sub)] = jnp.sum(D * eye, axis=2)


@functools.partial(jax.jit, static_argnames=("batch", "dim"))
def _tc_gather(idx, Wt, *, batch, dim):
    grid = batch // _GRP
    idx3 = idx.reshape(grid, 1, _GRP)
    grid_spec = pltpu.PrefetchScalarGridSpec(
        num_scalar_prefetch=1,
        grid=(grid,),
        in_specs=[
            pl.BlockSpec((1, 1, _GRP), lambda g, sref: (g, 0, 0)),
            pl.BlockSpec(memory_space=pltpu.HBM),
        ],
        out_specs=pl.BlockSpec((1, _GRP, dim), lambda g, sref: (g, 0, 0)),
        scratch_shapes=[
            pltpu.VMEM((2, _GRP, dim, 128), jnp.float32),
            pltpu.SemaphoreType.DMA((2, 4)),
        ],
    )
    out = pl.pallas_call(
        _tc_body,
        grid_spec=grid_spec,
        out_shape=jax.ShapeDtypeStruct((grid, _GRP, dim), jnp.float32),
        compiler_params=pltpu.CompilerParams(disable_bounds_checks=True),
    )(idx, idx3, Wt)
    return out.reshape(batch, dim)


_B_SC = 9984


def kernel(g, node_id_copy, W):
    batch = node_id_copy.shape[0]
    dim = W.shape[1]
    b_sc = min(_B_SC // 128 * 128, batch - batch % 128)
    if (batch - b_sc) % _GRP or batch - b_sc < _GRP or b_sc > batch - 384:
        b_sc = batch // 2 // (128 * _NW) * (128 * _NW)
    Wt = W.T
    out_sc_t = _gather(node_id_copy, Wt, b_sc=b_sc, dim=dim)
    out_tc = _tc_gather(node_id_copy[b_sc:], Wt, batch=batch - b_sc, dim=dim)
    return jnp.concatenate([out_sc_t.T, out_tc], axis=0)
